# Initial kernel scaffold; baseline (speedup 1.0000x reference)
import jax
import jax.numpy as jnp
from jax.experimental import pallas as pl


def kernel(a, emb):
    raise NotImplementedError("write your pallas kernel here")


# speedup vs baseline: 1.3027x; 1.3027x over previous
import functools

import jax
import jax.numpy as jnp
from jax import lax
from jax.experimental import pallas as pl
from jax.experimental.pallas import tpu as pltpu
from jax.experimental.pallas import tpu_sc as plsc

B = 16384
D = 64
BP = B // 2
DP = 2 * D

_info = plsc.get_sparse_core_info()
_NC, _NS, _L = _info.num_cores, _info.num_subcores, _info.num_lanes
_NW = _NC * _NS
_PPW = BP // _NW
_IPW = B // _NW

_mesh = plsc.VectorSubcoreMesh(core_axis_name="c", subcore_axis_name="s")


@functools.partial(
    pl.kernel,
    mesh=_mesh,
    out_type=jax.ShapeDtypeStruct((BP, DP), jnp.float32),
    compiler_params=pltpu.CompilerParams(needs_layout_passes=False),
    scratch_types=[
        pltpu.VMEM((_IPW,), jnp.int32),
        pltpu.VMEM((_PPW,), jnp.int32),
        pltpu.VMEM((_PPW, DP), jnp.float32),
        pltpu.SemaphoreType.DMA,
    ],
)
def _gather_kernel(pairs_hbm, idx_hbm, out_hbm, idx_v, pidx_v, rows_v, sem):
    wid = lax.axis_index("s") * _NC + lax.axis_index("c")
    pltpu.sync_copy(idx_hbm.at[pl.ds(wid * _IPW, _IPW)], idx_v)
    lane = lax.iota(jnp.int32, _L)
    for g in range(_PPW // _L):
        even = plsc.load_gather(idx_v, [lane * 2 + (2 * _L) * g])
        odd = plsc.load_gather(idx_v, [lane * 2 + ((2 * _L) * g + 1)])
        pidx_v[pl.ds(g * _L, _L)] = even * 4 + odd
    pltpu.async_copy(pairs_hbm.at[pidx_v], rows_v, sem).wait()
    pltpu.sync_copy(rows_v, out_hbm.at[pl.ds(wid * _PPW, _PPW)])


def kernel(a, emb):
    pairs = jnp.concatenate(
        [jnp.repeat(emb, 4, axis=0), jnp.tile(emb, (4, 1))], axis=-1
    )
    out = _gather_kernel(pairs, a.astype(jnp.int32))
    return out.reshape(B, 1, D)
